# baseline (speedup 1.0000x reference)
import jax
import jax.numpy as jnp
from jax.experimental import pallas as pl
from jax.experimental.pallas import tpu as pltpu

N = 50000
D = 128
T = 32
S = 32
B = 16
BN = 5000
NBLK = N // BN


def _ect_block_kernel(sc_ref, idx_ref, x_ref, v_ref, lin_ref, out_ref,
                      acc_ref, cnt_ref):
    i = pl.program_id(0)
    sc2 = sc_ref[0, 0].astype(jnp.float32) * jnp.float32(0.5)

    x = x_ref[...]
    v_s = v_ref[...] * sc2
    nh_t = jax.lax.dot_general(v_s, x, (((0,), (1,)), ((), ())),
                               preferred_element_type=jnp.float32)
    nh_all = jax.lax.broadcast_in_dim(nh_t, (S, T, BN), (1, 2))
    nh_all = nh_all.reshape(S * T, BN)
    lin_col = jax.lax.broadcast_in_dim(lin_ref[...] * sc2, (S, T, 1), (0, 2))
    lin_col = lin_col.reshape(S * T, 1)
    tanh_v = jnp.tanh(lin_col - nh_all).astype(jnp.bfloat16)

    idx_col = idx_ref[0].reshape(BN, 1)
    seg = jax.lax.broadcasted_iota(jnp.int32, (BN, B), 1)
    onehot = (idx_col == seg).astype(jnp.bfloat16)
    part = jax.lax.dot_general(tanh_v, onehot, (((1,), (0,)), ((), ())),
                               preferred_element_type=jnp.float32)
    ones = jnp.ones((8, BN), dtype=jnp.bfloat16)
    cnt = jax.lax.dot_general(ones, onehot, (((1,), (0,)), ((), ())),
                              preferred_element_type=jnp.float32)

    @pl.when(i == 0)
    def _init():
        acc_ref[...] = jnp.zeros_like(acc_ref)
        cnt_ref[...] = jnp.zeros_like(cnt_ref)

    acc_ref[...] += part
    cnt_ref[...] += cnt

    @pl.when(i == pl.num_programs(0) - 1)
    def _norm():
        tot = acc_ref[...] + cnt_ref[0:1, :]
        tot = tot / jnp.max(tot, axis=0, keepdims=True)
        out_ref[...] = tot.T


def kernel(x, index, v, lin, scale):
    sc = jnp.asarray(scale, dtype=jnp.int32).reshape(1, 1)
    lin2 = lin.astype(jnp.float32).reshape(S, 1)
    idx3 = index.astype(jnp.int32).reshape(NBLK, 1, BN)

    out = pl.pallas_call(
        _ect_block_kernel,
        grid=(NBLK,),
        in_specs=[
            pl.BlockSpec(memory_space=pltpu.SMEM),
            pl.BlockSpec((1, 1, BN), lambda i: (i, 0, 0)),
            pl.BlockSpec((BN, D), lambda i: (i, 0)),
            pl.BlockSpec((D, T), lambda i: (0, 0)),
            pl.BlockSpec((S, 1), lambda i: (0, 0)),
        ],
        out_specs=pl.BlockSpec((B, S * T), lambda i: (0, 0)),
        out_shape=jax.ShapeDtypeStruct((B, S * T), jnp.float32),
        scratch_shapes=[pltpu.VMEM((S * T, B), jnp.float32),
                        pltpu.VMEM((8, B), jnp.float32)],
        compiler_params=pltpu.CompilerParams(
            dimension_semantics=("arbitrary",)),
    )(sc, idx3, x, v, lin2)

    return out.reshape(B, S, T)


# speedup vs baseline: 1.0633x; 1.0633x over previous
import jax
import jax.numpy as jnp
from jax.experimental import pallas as pl
from jax.experimental.pallas import tpu as pltpu

N = 50000
D = 128
T = 32
S = 32
B = 16
BN = 5000
NBLK = N // BN


def _ect_block_kernel(sc_ref, idx_ref, x_ref, v_ref, lin_ref, out_ref,
                      acc_ref, cnt_ref):
    i = pl.program_id(0)
    sc2 = sc_ref[0, 0].astype(jnp.float32) * jnp.float32(0.5)

    x = x_ref[...]
    v_s = v_ref[...] * sc2
    nh_t = jax.lax.dot_general(v_s, x, (((0,), (1,)), ((), ())),
                               preferred_element_type=jnp.float32)
    nh_all = jax.lax.broadcast_in_dim(nh_t, (S, T, BN), (1, 2))
    nh_all = nh_all.reshape(S * T, BN)
    lin_col = jax.lax.broadcast_in_dim(lin_ref[...] * sc2, (S, T, 1), (0, 2))
    lin_col = lin_col.reshape(S * T, 1)
    tanh_v = jnp.tanh(lin_col - nh_all).astype(jnp.bfloat16)

    idx_row = idx_ref[0]
    seg = jax.lax.broadcasted_iota(jnp.int32, (B, BN), 0)
    onehot = (idx_row == seg).astype(jnp.bfloat16)
    part = jax.lax.dot_general(onehot, tanh_v, (((1,), (1,)), ((), ())),
                               preferred_element_type=jnp.float32)
    ones = jnp.ones((BN, 8), dtype=jnp.bfloat16)
    cnt = jax.lax.dot_general(onehot, ones, (((1,), (0,)), ((), ())),
                              preferred_element_type=jnp.float32)

    @pl.when(i == 0)
    def _init():
        acc_ref[...] = jnp.zeros_like(acc_ref)
        cnt_ref[...] = jnp.zeros_like(cnt_ref)

    acc_ref[...] += part
    cnt_ref[...] += cnt

    @pl.when(i == pl.num_programs(0) - 1)
    def _norm():
        tot = acc_ref[...] + cnt_ref[:, 0:1]
        out_ref[...] = tot / jnp.max(tot, axis=1, keepdims=True)


def kernel(x, index, v, lin, scale):
    sc = jnp.asarray(scale, dtype=jnp.int32).reshape(1, 1)
    lin2 = lin.astype(jnp.float32).reshape(S, 1)
    idx3 = index.astype(jnp.int32).reshape(NBLK, 1, BN)

    out = pl.pallas_call(
        _ect_block_kernel,
        grid=(NBLK,),
        in_specs=[
            pl.BlockSpec(memory_space=pltpu.SMEM),
            pl.BlockSpec((1, 1, BN), lambda i: (i, 0, 0)),
            pl.BlockSpec((BN, D), lambda i: (i, 0)),
            pl.BlockSpec((D, T), lambda i: (0, 0)),
            pl.BlockSpec((S, 1), lambda i: (0, 0)),
        ],
        out_specs=pl.BlockSpec((B, S * T), lambda i: (0, 0)),
        out_shape=jax.ShapeDtypeStruct((B, S * T), jnp.float32),
        scratch_shapes=[pltpu.VMEM((B, S * T), jnp.float32),
                        pltpu.VMEM((B, 8), jnp.float32)],
        compiler_params=pltpu.CompilerParams(
            dimension_semantics=("arbitrary",)),
    )(sc, idx3, x, v, lin2)

    return out.reshape(B, S, T)
